# baseline (speedup 1.0000x reference)
import functools

import jax
import jax.numpy as jnp
from jax.experimental import pallas as pl
from jax.experimental.pallas import tpu as pltpu

_TOKENS_PER_TILE = 2048
_LANES = 128


def _round_up(x, m):
    return (x + m - 1) // m * m


def _gather_tile_kernel(ids_ref, table_ref, o_ref, *, tokens, p):
    base = pl.program_id(0) * tokens
    for mi in range(tokens):
        idx = pl.multiple_of(ids_ref[base + mi], p)
        o_ref[pl.ds(mi * p, p), :] = table_ref[pl.ds(idx, p), :]


def kernel(sentence, embed_table):
    B, S = sentence.shape
    V, E = embed_table.shape
    T = _TOKENS_PER_TILE
    p = E // _LANES

    flat = sentence.reshape(-1).astype(jnp.int32)
    N = flat.shape[0]
    N_pad = _round_up(N, T)
    if N_pad != N:
        flat = jnp.pad(flat, (0, N_pad - N))
    ids = flat * p

    table_v = embed_table.reshape(V * p, _LANES)
    grid = (N_pad // T,)

    vmem_bytes = V * E * 4 + 4 * T * E * 4 + (4 << 20)

    out = pl.pallas_call(
        functools.partial(_gather_tile_kernel, tokens=T, p=p),
        out_shape=jax.ShapeDtypeStruct((N_pad * p, _LANES), embed_table.dtype),
        grid_spec=pltpu.PrefetchScalarGridSpec(
            num_scalar_prefetch=1,
            grid=grid,
            in_specs=[
                pl.BlockSpec((V * p, _LANES), lambda i, ids: (0, 0)),
            ],
            out_specs=pl.BlockSpec((T * p, _LANES), lambda i, ids: (i, 0)),
        ),
        compiler_params=pltpu.CompilerParams(
            dimension_semantics=("parallel",),
            vmem_limit_bytes=vmem_bytes,
        ),
    )(ids, table_v)

    return out[: N * p].reshape(B, S, E)


# speedup vs baseline: 1.9968x; 1.9968x over previous
import functools

import jax
import jax.numpy as jnp
from jax.experimental import pallas as pl
from jax.experimental.pallas import tpu as pltpu

_TOKENS_PER_TILE = 2048
_LANES = 128


def _round_up(x, m):
    return (x + m - 1) // m * m


def _gather_tile_kernel(ids_ref, table_ref, o_ref, *, tokens, p):
    base = pl.program_id(0) * tokens
    for mi in range(tokens):
        idx = pl.multiple_of(ids_ref[base + mi], p)
        slab = table_ref[pl.ds(idx, p), :]
        o_ref[mi, :] = slab.reshape(p * _LANES)


def kernel(sentence, embed_table):
    B, S = sentence.shape
    V, E = embed_table.shape
    T = _TOKENS_PER_TILE
    p = E // _LANES

    flat = sentence.reshape(-1).astype(jnp.int32)
    N = flat.shape[0]
    N_pad = _round_up(N, T)
    if N_pad != N:
        flat = jnp.pad(flat, (0, N_pad - N))
    ids = flat * p

    table_v = embed_table.reshape(V * p, _LANES)
    grid = (N_pad // T,)

    vmem_bytes = V * E * 4 + 4 * T * E * 4 + (4 << 20)

    out = pl.pallas_call(
        functools.partial(_gather_tile_kernel, tokens=T, p=p),
        out_shape=jax.ShapeDtypeStruct((N_pad, E), embed_table.dtype),
        grid_spec=pltpu.PrefetchScalarGridSpec(
            num_scalar_prefetch=1,
            grid=grid,
            in_specs=[
                pl.BlockSpec((V * p, _LANES), lambda i, ids: (0, 0)),
            ],
            out_specs=pl.BlockSpec((T, E), lambda i, ids: (i, 0)),
        ),
        compiler_params=pltpu.CompilerParams(
            dimension_semantics=("parallel",),
            vmem_limit_bytes=vmem_bytes,
        ),
    )(ids, table_v)

    return out[:N].reshape(B, S, E)
